# baseline (speedup 1.0000x reference)
import functools

import jax
import jax.numpy as jnp
from jax import lax
from jax.experimental import pallas as pl
from jax.experimental.pallas import tpu as pltpu
from jax.experimental.pallas import tpu_sc as plsc

_BETA = 0.25
_K = 1024
_C = 256
_R = 4096
_N = 32768

_NC = 2
_NS = 16
_NW = _NC * _NS
_BPW = _N // _NW
_CH = 128


def _vq_body(z_ref, embTn_ref, es_ref, inds_ref, loss_ref):
    step = pl.program_id(0)

    zb = z_ref[...]
    dotn = jax.lax.dot_general(
        zb, embTn_ref[...], (((1,), (0,)), ((), ())),
        preferred_element_type=jnp.float32)
    zs_col = jnp.sum(zb * zb, axis=1, keepdims=True)
    d = (zs_col + es_ref[...]) + dotn

    m = jnp.min(d, axis=1, keepdims=True)
    iota_k = jax.lax.broadcasted_iota(jnp.int32, (_R, _K), 1)
    idx = jnp.min(jnp.where(d == m, iota_k, _K), axis=1, keepdims=True)
    inds_ref[...] = idx

    @pl.when(step == 0)
    def _init():
        loss_ref[...] = jnp.zeros_like(loss_ref)

    loss_ref[...] += jnp.sum(m, axis=(0, 1), keepdims=True).reshape(1, 1)


_sc_mesh = plsc.VectorSubcoreMesh(core_axis_name="c", subcore_axis_name="s")


@functools.partial(
    pl.kernel,
    mesh=_sc_mesh,
    out_type=jax.ShapeDtypeStruct((_N, _C), jnp.float32),
    scratch_types=[
        pltpu.VMEM((_CH,), jnp.int32),
        pltpu.VMEM((_CH, _C), jnp.float32),
        pltpu.SemaphoreType.DMA,
    ],
)
def _gather_sc(table_hbm, idx_hbm, out_hbm, idx_v, rows_v, sem):
    wid = lax.axis_index("s") * _NC + lax.axis_index("c")
    base = wid * _BPW

    def body(i, carry):
        off = base + i * _CH
        pltpu.sync_copy(idx_hbm.at[pl.ds(off, _CH)], idx_v)
        pltpu.async_copy(table_hbm.at[idx_v], rows_v, sem).wait()
        pltpu.sync_copy(rows_v, out_hbm.at[pl.ds(off, _CH)])
        return carry

    lax.fori_loop(0, _BPW // _CH, body, 0)


@functools.partial(jax.jit, static_argnames=())
def kernel(z, embedding):
    B, C, D, H, W = z.shape
    K = embedding.shape[0]
    xp = jnp.transpose(z, (0, 2, 3, 4, 1)).reshape(-1, C)
    es = jnp.sum(embedding ** 2, axis=1).reshape(1, K)
    embTn = embedding.T * -2.0

    inds2, loss_acc = pl.pallas_call(
        _vq_body,
        grid=(_N // _R,),
        in_specs=[
            pl.BlockSpec((_R, _C), lambda i: (i, 0)),
            pl.BlockSpec((_C, _K), lambda i: (0, 0)),
            pl.BlockSpec((1, _K), lambda i: (0, 0)),
        ],
        out_specs=[
            pl.BlockSpec((_R, 1), lambda i: (i, 0)),
            pl.BlockSpec((1, 1), lambda i: (0, 0)),
        ],
        out_shape=[
            jax.ShapeDtypeStruct((_N, 1), jnp.int32),
            jax.ShapeDtypeStruct((1, 1), jnp.float32),
        ],
    )(xp, embTn, es)

    idx_flat = inds2.reshape(_N)
    zq2 = _gather_sc(embedding, idx_flat)

    z_q_out = jnp.transpose(zq2.reshape(B, D, H, W, C), (0, 4, 1, 2, 3))
    inds = inds2.reshape(B, D, H, W)
    loss = loss_acc[0, 0] * (_BETA / (B * D * H * W * C))
    return (z_q_out, inds, loss)


# speedup vs baseline: 1.0540x; 1.0540x over previous
import functools

import jax
import jax.numpy as jnp
from jax import lax
from jax.experimental import pallas as pl
from jax.experimental.pallas import tpu as pltpu
from jax.experimental.pallas import tpu_sc as plsc

_BETA = 0.25
_K = 1024
_C = 256
_R = 8192
_N = 32768

_NC = 2
_NS = 16
_NW = _NC * _NS
_BPW = _N // _NW
_CH = 256


def _vq_body(z_ref, embTn_ref, es_ref, inds_ref, loss_ref):
    step = pl.program_id(0)

    zb = z_ref[...]
    dotn = jax.lax.dot_general(
        zb, embTn_ref[...], (((1,), (0,)), ((), ())),
        preferred_element_type=jnp.float32)
    zs_col = jnp.sum(zb * zb, axis=1, keepdims=True)
    d = (zs_col + es_ref[...]) + dotn

    m = jnp.min(d, axis=1, keepdims=True)
    iota_k = jax.lax.broadcasted_iota(jnp.int32, (_R, _K), 1)
    idx = jnp.min(jnp.where(d == m, iota_k, _K), axis=1, keepdims=True)
    inds_ref[...] = idx

    @pl.when(step == 0)
    def _init():
        loss_ref[...] = jnp.zeros_like(loss_ref)

    loss_ref[...] += jnp.sum(m, axis=(0, 1), keepdims=True).reshape(1, 1)


_sc_mesh = plsc.VectorSubcoreMesh(core_axis_name="c", subcore_axis_name="s")


@functools.partial(
    pl.kernel,
    mesh=_sc_mesh,
    out_type=jax.ShapeDtypeStruct((_N, _C), jnp.float32),
    scratch_types=[
        pltpu.VMEM((_CH,), jnp.int32),
        pltpu.VMEM((_CH, _C), jnp.float32),
        pltpu.SemaphoreType.DMA,
    ],
)
def _gather_sc(table_hbm, idx_hbm, out_hbm, idx_v, rows_v, sem):
    wid = lax.axis_index("s") * _NC + lax.axis_index("c")
    base = wid * _BPW

    def body(i, carry):
        off = base + i * _CH
        pltpu.sync_copy(idx_hbm.at[pl.ds(off, _CH)], idx_v)
        pltpu.async_copy(table_hbm.at[idx_v], rows_v, sem).wait()
        pltpu.sync_copy(rows_v, out_hbm.at[pl.ds(off, _CH)])
        return carry

    lax.fori_loop(0, _BPW // _CH, body, 0)


@functools.partial(jax.jit, static_argnames=())
def kernel(z, embedding):
    B, C, D, H, W = z.shape
    K = embedding.shape[0]
    xp = jnp.transpose(z, (0, 2, 3, 4, 1)).reshape(-1, C)
    es = jnp.sum(embedding ** 2, axis=1).reshape(1, K)
    embTn = embedding.T * -2.0

    inds2, loss_acc = pl.pallas_call(
        _vq_body,
        grid=(_N // _R,),
        in_specs=[
            pl.BlockSpec((_R, _C), lambda i: (i, 0)),
            pl.BlockSpec((_C, _K), lambda i: (0, 0)),
            pl.BlockSpec((1, _K), lambda i: (0, 0)),
        ],
        out_specs=[
            pl.BlockSpec((_R, 1), lambda i: (i, 0)),
            pl.BlockSpec((1, 1), lambda i: (0, 0)),
        ],
        out_shape=[
            jax.ShapeDtypeStruct((_N, 1), jnp.int32),
            jax.ShapeDtypeStruct((1, 1), jnp.float32),
        ],
    )(xp, embTn, es)

    idx_flat = inds2.reshape(_N)
    zq2 = _gather_sc(embedding, idx_flat)

    z_q_out = jnp.transpose(zq2.reshape(B, D, H, W, C), (0, 4, 1, 2, 3))
    inds = inds2.reshape(B, D, H, W)
    loss = loss_acc[0, 0] * (_BETA / (B * D * H * W * C))
    return (z_q_out, inds, loss)
